# baseline (speedup 1.0000x reference)
import functools

import numpy as np
import jax
import jax.numpy as jnp
from jax import lax
from jax.experimental import pallas as pl
from jax.experimental.pallas import tpu as pltpu
from jax.experimental.pallas import tpu_sc as plsc

_NF = 26
_VOCAB = 100000
_DE = 32
_DM = 128
_B = 1024
_L = 50
_BL = _B * _L
_FAN_IN = _NF * _DE
_NGRP = _BL // 8

_NC = 2
_NS = 16
_NW = _NC * _NS
_GRP_W = _NGRP // _NW
_IDX_W = _GRP_W * 8 * _NF
_G = 128
_CG = 5

_NF_A = 16
_NF_B = 10
_NT_A = 4
_NT_B = 3
_RG_A = 8 * 4 * _NT_A
_RG_B = 8 * 4 * _NT_B


def _patterns(j_lo, j_hi, i_base):
    rg = 8 * 4 * (j_hi - j_lo)
    perm = np.zeros(rg, dtype=np.int32)
    off = np.zeros(rg, dtype=np.int32)
    for j in range(j_lo, j_hi):
        for r in range(8):
            for p in range(4):
                i = 4 * j + p
                k = (j - j_lo) * 32 + r * 4 + p
                if i < _NF:
                    perm[k] = r * _NF + i
                    off[k] = 4 * (i - i_base) * _VOCAB
                else:
                    perm[k] = r * _NF + i_base
                    off[k] = 0
    return perm, off


_PERM_A, _OFF_A = _patterns(0, _NT_A, 0)
_PERM_B, _OFF_B = _patterns(_NT_A, 7, _NF_A)

_mesh = plsc.VectorSubcoreMesh(core_axis_name="c", subcore_axis_name="s")


def _gather_body(rg, samp_hbm, table_hbm, perm_hbm, off_hbm, out_hbm,
                 samp_v, idxp_v, perm_v, off_v, rows_v, gsem):
    rows_w = _GRP_W * rg
    nchunk = rows_w // (_CG * _G)
    wid = lax.axis_index("s") * _NC + lax.axis_index("c")
    pltpu.sync_copy(perm_hbm, perm_v)
    pltpu.sync_copy(off_hbm, off_v)
    pltpu.sync_copy(samp_hbm.at[wid], samp_v)

    @pl.loop(0, _GRP_W)
    def _(g):
        sb = g * (8 * _NF)
        tb = g * rg
        for s in range(rg // 16):
            pv = perm_v[pl.ds(s * 16, 16)] + sb
            vals = plsc.load_gather(samp_v, [pv])
            idxp_v[pl.ds(tb + s * 16, 16)] = (
                vals * 4 + off_v[pl.ds(s * 16, 16)]
            )

    base = wid * rows_w

    @pl.loop(0, nchunk)
    def _(c):
        copies = []
        for g in range(_CG):
            copies.append(
                pltpu.async_copy(
                    table_hbm.at[idxp_v.at[pl.ds((c * _CG + g) * _G, _G)]],
                    rows_v.at[pl.ds(g * _G, _G)],
                    gsem,
                )
            )
        for cp in copies:
            cp.wait()
        pltpu.sync_copy(
            rows_v, out_hbm.at[pl.ds(base + c * _CG * _G, _CG * _G)]
        )


def _sc_gather(samp_rs, tables_flat, perm, off, rg, name):
    rows_w = _GRP_W * rg
    k = functools.partial(
        pl.kernel,
        mesh=_mesh,
        compiler_params=pltpu.CompilerParams(
            use_tc_tiling_on_sc=False, needs_layout_passes=False
        ),
        out_type=jax.ShapeDtypeStruct((_NGRP * rg, _DE), jnp.float32),
        name=name,
        scratch_types=[
            pltpu.VMEM((_IDX_W,), jnp.int32),
            pltpu.VMEM((rows_w,), jnp.int32),
            pltpu.VMEM((rg,), jnp.int32),
            pltpu.VMEM((rg,), jnp.int32),
            pltpu.VMEM((_CG * _G, _DE), jnp.float32),
            pltpu.SemaphoreType.DMA,
        ],
    )(functools.partial(_gather_body, rg))
    return k(samp_rs, tables_flat, perm, off)


_BG = 256


def _mm_body(xa_ref, xb_ref, w_ref, b_ref, o_ref):
    acc = jnp.broadcast_to(b_ref[...], (_BG * 8, _DM))
    for j in range(_NT_A):
        xj = xa_ref[:, j].reshape(_BG * 8, _DM)
        acc = acc + jnp.dot(xj, w_ref[j], preferred_element_type=jnp.float32)
    for j in range(_NT_B):
        xj = xb_ref[:, j].reshape(_BG * 8, _DM)
        acc = acc + jnp.dot(
            xj, w_ref[_NT_A + j], preferred_element_type=jnp.float32
        )
    o_ref[...] = acc


def _mm(xa, xb, w4, b2):
    return pl.pallas_call(
        _mm_body,
        grid=(_NGRP // _BG,),
        in_specs=[
            pl.BlockSpec((_BG, _NT_A, 8, _DM), lambda i: (i, 0, 0, 0)),
            pl.BlockSpec((_BG, _NT_B, 8, _DM), lambda i: (i, 0, 0, 0)),
            pl.BlockSpec((7, _DM, _DM), lambda i: (0, 0, 0)),
            pl.BlockSpec((1, _DM), lambda i: (0, 0)),
        ],
        out_specs=pl.BlockSpec((_BG * 8, _DM), lambda i: (i, 0)),
        out_shape=jax.ShapeDtypeStruct((_BL, _DM), jnp.float32),
    )(xa, xb, w4, b2)


def kernel(sample, tables, W, b):
    samp_rs = sample.reshape(_NW, _IDX_W)
    pad = ((0, 0), (0, 0), (0, 128 - _DE))
    ta = jnp.pad(tables[:_NF_A], pad).reshape(4 * _NF_A * _VOCAB, _DE)
    tb = jnp.pad(tables[_NF_A:], pad).reshape(4 * _NF_B * _VOCAB, _DE)
    ga = _sc_gather(
        samp_rs, ta, jnp.asarray(_PERM_A), jnp.asarray(_OFF_A), _RG_A,
        "sc_gather_a",
    )
    gb = _sc_gather(
        samp_rs, tb, jnp.asarray(_PERM_B), jnp.asarray(_OFF_B), _RG_B,
        "sc_gather_b",
    )
    xa = ga.reshape(_NGRP, _NT_A, 8, _DM)
    xb = gb.reshape(_NGRP, _NT_B, 8, _DM)
    w4 = (
        jnp.zeros((7 * _DM, _DM), jnp.float32)
        .at[:_FAN_IN]
        .set(W)
        .reshape(7, _DM, _DM)
    )
    out = _mm(xa, xb, w4, b.reshape(1, _DM))
    return out.reshape(_B, _L, _DM)


# speedup vs baseline: 1.0001x; 1.0001x over previous
import functools

import numpy as np
import jax
import jax.numpy as jnp
from jax import lax
from jax.experimental import pallas as pl
from jax.experimental.pallas import tpu as pltpu
from jax.experimental.pallas import tpu_sc as plsc

_NF = 26
_VOCAB = 100000
_DE = 32
_DM = 128
_B = 1024
_L = 50
_BL = _B * _L
_FAN_IN = _NF * _DE
_NGRP = _BL // 8

_NC = 2
_NS = 16
_NW = _NC * _NS
_GRP_W = _NGRP // _NW
_IDX_W = _GRP_W * 8 * _NF
_G = 128
_CG = 5

_NF_A = 16
_NF_B = 10
_NT_A = 4
_NT_B = 3
_RG_A = 8 * 4 * _NT_A
_RG_B = 8 * 4 * _NT_B


def _patterns(j_lo, j_hi, i_base):
    rg = 8 * 4 * (j_hi - j_lo)
    perm = np.zeros(rg, dtype=np.int32)
    off = np.zeros(rg, dtype=np.int32)
    for j in range(j_lo, j_hi):
        for r in range(8):
            for p in range(4):
                i = 4 * j + p
                k = (j - j_lo) * 32 + r * 4 + p
                if i < _NF:
                    perm[k] = r * _NF + i
                    off[k] = 4 * (i - i_base) * _VOCAB
                else:
                    perm[k] = r * _NF + i_base
                    off[k] = 0
    return perm, off


_PERM_A, _OFF_A = _patterns(0, _NT_A, 0)
_PERM_B, _OFF_B = _patterns(_NT_A, 7, _NF_A)

_mesh = plsc.VectorSubcoreMesh(core_axis_name="c", subcore_axis_name="s")


def _gather_body(rg, samp_hbm, table_hbm, perm_hbm, off_hbm, out_hbm,
                 samp_v, idxp_v, perm_v, off_v, rows_v, gsem):
    rows_w = _GRP_W * rg
    nchunk = rows_w // (_CG * _G)
    wid = lax.axis_index("s") * _NC + lax.axis_index("c")
    pltpu.sync_copy(perm_hbm, perm_v)
    pltpu.sync_copy(off_hbm, off_v)
    pltpu.sync_copy(samp_hbm.at[wid], samp_v)

    @pl.loop(0, _GRP_W)
    def _(g):
        sb = g * (8 * _NF)
        tb = g * rg
        for s in range(rg // 16):
            pv = perm_v[pl.ds(s * 16, 16)] + sb
            vals = plsc.load_gather(samp_v, [pv])
            idxp_v[pl.ds(tb + s * 16, 16)] = (
                vals * 4 + off_v[pl.ds(s * 16, 16)]
            )

    base = wid * rows_w

    @pl.loop(0, nchunk)
    def _(c):
        copies = []
        for g in range(_CG):
            copies.append(
                pltpu.async_copy(
                    table_hbm.at[idxp_v.at[pl.ds((c * _CG + g) * _G, _G)]],
                    rows_v.at[pl.ds(g * _G, _G)],
                    gsem,
                )
            )
        for cp in copies:
            cp.wait()
        pltpu.sync_copy(
            rows_v, out_hbm.at[pl.ds(base + c * _CG * _G, _CG * _G)]
        )


def _sc_gather(samp_rs, tables_flat, perm, off, rg, name):
    rows_w = _GRP_W * rg
    k = functools.partial(
        pl.kernel,
        mesh=_mesh,
        compiler_params=pltpu.CompilerParams(
            use_tc_tiling_on_sc=False, needs_layout_passes=False
        ),
        out_type=jax.ShapeDtypeStruct((_NGRP * rg, _DE), jnp.float32),
        name=name,
        scratch_types=[
            pltpu.VMEM((_IDX_W,), jnp.int32),
            pltpu.VMEM((rows_w,), jnp.int32),
            pltpu.VMEM((rg,), jnp.int32),
            pltpu.VMEM((rg,), jnp.int32),
            pltpu.VMEM((_CG * _G, _DE), jnp.float32),
            pltpu.SemaphoreType.DMA,
        ],
    )(functools.partial(_gather_body, rg))
    return k(samp_rs, tables_flat, perm, off)


_BG = 256


def _mm_body(xa_ref, xb_ref, w_ref, b_ref, o_ref):
    acc = jnp.broadcast_to(b_ref[...], (_BG * 8, _DM))
    for j in range(_NT_A):
        xj = xa_ref[:, j].reshape(_BG * 8, _DM)
        acc = acc + jnp.dot(xj, w_ref[j], preferred_element_type=jnp.float32)
    for j in range(_NT_B):
        xj = xb_ref[:, j].reshape(_BG * 8, _DM)
        acc = acc + jnp.dot(
            xj, w_ref[_NT_A + j], preferred_element_type=jnp.float32
        )
    o_ref[...] = acc


def _mm(xa, xb, w4, b2):
    return pl.pallas_call(
        _mm_body,
        grid=(_NGRP // _BG,),
        in_specs=[
            pl.BlockSpec((_BG, _NT_A, 8, _DM), lambda i: (i, 0, 0, 0)),
            pl.BlockSpec((_BG, _NT_B, 8, _DM), lambda i: (i, 0, 0, 0)),
            pl.BlockSpec((7, _DM, _DM), lambda i: (0, 0, 0)),
            pl.BlockSpec((1, _DM), lambda i: (0, 0)),
        ],
        out_specs=pl.BlockSpec((_BG * 8, _DM), lambda i: (i, 0)),
        out_shape=jax.ShapeDtypeStruct((_BL, _DM), jnp.float32),
    )(xa, xb, w4, b2)


def kernel(sample, tables, W, b):
    samp_rs = sample.reshape(_NW, _IDX_W)
    pad = ((0, 0), (0, 0), (0, 128 - _DE))
    ta = jnp.pad(tables[:_NF_A], pad).reshape(4 * _NF_A * _VOCAB, _DE)
    ga = _sc_gather(
        samp_rs, ta, jnp.asarray(_PERM_A), jnp.asarray(_OFF_A), _RG_A,
        "sc_gather_a",
    )
    tb = jnp.pad(tables[_NF_A:], pad).reshape(4 * _NF_B * _VOCAB, _DE)
    gb = _sc_gather(
        samp_rs, tb, jnp.asarray(_PERM_B), jnp.asarray(_OFF_B), _RG_B,
        "sc_gather_b",
    )
    xa = ga.reshape(_NGRP, _NT_A, 8, _DM)
    xb = gb.reshape(_NGRP, _NT_B, 8, _DM)
    w4 = (
        jnp.zeros((7 * _DM, _DM), jnp.float32)
        .at[:_FAN_IN]
        .set(W)
        .reshape(7, _DM, _DM)
    )
    out = _mm(xa, xb, w4, b.reshape(1, _DM))
    return out.reshape(_B, _L, _DM)


# speedup vs baseline: 1.0808x; 1.0807x over previous
import functools

import numpy as np
import jax
import jax.numpy as jnp
from jax import lax
from jax.experimental import pallas as pl
from jax.experimental.pallas import tpu as pltpu
from jax.experimental.pallas import tpu_sc as plsc

_NF = 26
_VOCAB = 100000
_DE = 32
_DM = 128
_B = 1024
_L = 50
_BL = _B * _L
_FAN_IN = _NF * _DE
_FAN_PAD = 896
_NTILE = 7
_NGRP = _BL // 8

_NC = 2
_NS = 16
_NW = _NC * _NS
_GRP_W = _NGRP // _NW
_IDX_W = _GRP_W * 8 * _NF
_ROWS_GRP = 8 * 4 * _NTILE
_ROWS_W = _GRP_W * _ROWS_GRP
_TOT_ROWS = _NGRP * _ROWS_GRP
_G = 128
_CG = 5
_CHUNK = _CG * _G
_NCHUNK = _ROWS_W // _CHUNK

_PERM_NP = np.zeros(_ROWS_GRP, dtype=np.int32)
_OFF_NP = np.zeros(_ROWS_GRP, dtype=np.int32)
for _j in range(_NTILE):
    for _r in range(8):
        for _p in range(4):
            _i = 4 * _j + _p
            _k = _j * 32 + _r * 4 + _p
            if _i < _NF:
                _PERM_NP[_k] = _r * _NF + _i
                _OFF_NP[_k] = 4 * _i * _VOCAB
            else:
                _PERM_NP[_k] = _r * _NF
                _OFF_NP[_k] = 0

_mesh = plsc.VectorSubcoreMesh(core_axis_name="c", subcore_axis_name="s")


def _gather_body(samp_hbm, table_hbm, perm_hbm, off_hbm, out_hbm,
                 samp_v, idxp_v, perm_v, off_v, rows_v, gsem):
    wid = lax.axis_index("s") * _NC + lax.axis_index("c")
    pltpu.sync_copy(perm_hbm, perm_v)
    pltpu.sync_copy(off_hbm, off_v)
    pltpu.sync_copy(samp_hbm.at[wid], samp_v)

    @pl.loop(0, _GRP_W)
    def _(g):
        sb = g * (8 * _NF)
        tb = g * _ROWS_GRP
        for s in range(_ROWS_GRP // 16):
            pv = perm_v[pl.ds(s * 16, 16)] + sb
            vals = plsc.load_gather(samp_v, [pv])
            idxp_v[pl.ds(tb + s * 16, 16)] = (
                vals * 4 + off_v[pl.ds(s * 16, 16)]
            )

    base = wid * _ROWS_W

    @pl.loop(0, _NCHUNK)
    def _(c):
        copies = []
        for g in range(_CG):
            copies.append(
                pltpu.async_copy(
                    table_hbm.at[idxp_v.at[pl.ds((c * _CG + g) * _G, _G)]],
                    rows_v.at[pl.ds(g * _G, _G)],
                    gsem,
                )
            )
        for cp in copies:
            cp.wait()
        pltpu.sync_copy(rows_v, out_hbm.at[pl.ds(base + c * _CHUNK, _CHUNK)])


def _sc_gather(samp_rs, tables_flat, perm, off):
    k = functools.partial(
        pl.kernel,
        mesh=_mesh,
        compiler_params=pltpu.CompilerParams(
            use_tc_tiling_on_sc=False, needs_layout_passes=False
        ),
        out_type=jax.ShapeDtypeStruct((_TOT_ROWS, _DE), jnp.float32),
        name="sc_gather",
        scratch_types=[
            pltpu.VMEM((_IDX_W,), jnp.int32),
            pltpu.VMEM((_ROWS_W,), jnp.int32),
            pltpu.VMEM((_ROWS_GRP,), jnp.int32),
            pltpu.VMEM((_ROWS_GRP,), jnp.int32),
            pltpu.VMEM((_CHUNK, _DE), jnp.float32),
            pltpu.SemaphoreType.DMA,
        ],
    )(_gather_body)
    return k(samp_rs, tables_flat, perm, off)


_BG = 256


def _mm_body(x_ref, w_ref, b_ref, o_ref):
    acc = jnp.broadcast_to(b_ref[...], (_BG * 8, _DM))
    for j in range(_NTILE):
        xj = x_ref[:, j].reshape(_BG * 8, _DM)
        acc = acc + jnp.dot(xj, w_ref[j], preferred_element_type=jnp.float32)
    o_ref[...] = acc


def _mm(x4d, w4, b2):
    return pl.pallas_call(
        _mm_body,
        grid=(_NGRP // _BG,),
        in_specs=[
            pl.BlockSpec((_BG, _NTILE, 8, _DM), lambda i: (i, 0, 0, 0)),
            pl.BlockSpec((_NTILE, _DM, _DM), lambda i: (0, 0, 0)),
            pl.BlockSpec((1, _DM), lambda i: (0, 0)),
        ],
        out_specs=pl.BlockSpec((_BG * 8, _DM), lambda i: (i, 0)),
        out_shape=jax.ShapeDtypeStruct((_BL, _DM), jnp.float32),
    )(x4d, w4, b2)


def kernel(sample, tables, W, b):
    samp_rs = sample.reshape(_NW, _IDX_W)
    tpad = jnp.pad(tables, ((0, 0), (0, 0), (0, 128 - _DE)))
    tables_flat = tpad.reshape(4 * _NF * _VOCAB, _DE)
    gathered = _sc_gather(
        samp_rs, tables_flat, jnp.asarray(_PERM_NP), jnp.asarray(_OFF_NP)
    )
    x4d = gathered.reshape(_NGRP, _NTILE, 8, _DM)
    w4 = (
        jnp.zeros((_FAN_PAD, _DM), jnp.float32)
        .at[:_FAN_IN]
        .set(W)
        .reshape(_NTILE, _DM, _DM)
    )
    out = _mm(x4d, w4, b.reshape(1, _DM))
    return out.reshape(_B, _L, _DM)


# speedup vs baseline: 1.1036x; 1.0211x over previous
import functools

import numpy as np
import jax
import jax.numpy as jnp
from jax import lax
from jax.experimental import pallas as pl
from jax.experimental.pallas import tpu as pltpu
from jax.experimental.pallas import tpu_sc as plsc

_NF = 26
_VOCAB = 100000
_DE = 32
_DM = 128
_B = 1024
_L = 50
_BL = _B * _L
_FAN_IN = _NF * _DE
_FAN_PAD = 896
_NTILE = 7
_NGRP = _BL // 8

_NC = 2
_NS = 16
_NW = _NC * _NS
_GRP_W = _NGRP // _NW
_IDX_W = _GRP_W * 8 * _NF
_ROWS_GRP = 8 * 4 * _NTILE
_ROWS_W = _GRP_W * _ROWS_GRP
_TOT_ROWS = _NGRP * _ROWS_GRP
_G = 128
_CG = 10
_CHUNK = _CG * _G
_NCHUNK = _ROWS_W // _CHUNK

_PERM_NP = np.zeros(_ROWS_GRP, dtype=np.int32)
_OFF_NP = np.zeros(_ROWS_GRP, dtype=np.int32)
for _j in range(_NTILE):
    for _r in range(8):
        for _p in range(4):
            _i = 4 * _j + _p
            _k = _j * 32 + _r * 4 + _p
            if _i < _NF:
                _PERM_NP[_k] = _r * _NF + _i
                _OFF_NP[_k] = 4 * _i * _VOCAB
            else:
                _PERM_NP[_k] = _r * _NF
                _OFF_NP[_k] = 0

_mesh = plsc.VectorSubcoreMesh(core_axis_name="c", subcore_axis_name="s")


def _gather_body(samp_hbm, table_hbm, perm_hbm, off_hbm, out_hbm,
                 samp_v, idxp_v, perm_v, off_v, rows_v, gsem):
    wid = lax.axis_index("s") * _NC + lax.axis_index("c")
    pltpu.sync_copy(perm_hbm, perm_v)
    pltpu.sync_copy(off_hbm, off_v)
    pltpu.sync_copy(samp_hbm.at[wid], samp_v)

    @pl.loop(0, _GRP_W)
    def _(g):
        sb = g * (8 * _NF)
        tb = g * _ROWS_GRP
        for s in range(_ROWS_GRP // 16):
            pv = perm_v[pl.ds(s * 16, 16)] + sb
            vals = plsc.load_gather(samp_v, [pv])
            idxp_v[pl.ds(tb + s * 16, 16)] = (
                vals * 4 + off_v[pl.ds(s * 16, 16)]
            )

    base = wid * _ROWS_W

    @pl.loop(0, _NCHUNK)
    def _(c):
        copies = []
        for g in range(_CG):
            copies.append(
                pltpu.async_copy(
                    table_hbm.at[idxp_v.at[pl.ds((c * _CG + g) * _G, _G)]],
                    rows_v.at[pl.ds(g * _G, _G)],
                    gsem,
                )
            )
        for cp in copies:
            cp.wait()
        pltpu.sync_copy(rows_v, out_hbm.at[pl.ds(base + c * _CHUNK, _CHUNK)])


def _sc_gather(samp_rs, tables_flat, perm, off):
    k = functools.partial(
        pl.kernel,
        mesh=_mesh,
        compiler_params=pltpu.CompilerParams(
            use_tc_tiling_on_sc=False, needs_layout_passes=False
        ),
        out_type=jax.ShapeDtypeStruct((_TOT_ROWS, _DE), jnp.float32),
        name="sc_gather",
        scratch_types=[
            pltpu.VMEM((_IDX_W,), jnp.int32),
            pltpu.VMEM((_ROWS_W,), jnp.int32),
            pltpu.VMEM((_ROWS_GRP,), jnp.int32),
            pltpu.VMEM((_ROWS_GRP,), jnp.int32),
            pltpu.VMEM((_CHUNK, _DE), jnp.float32),
            pltpu.SemaphoreType.DMA,
        ],
    )(_gather_body)
    return k(samp_rs, tables_flat, perm, off)


_BG = 256


def _mm_body(x_ref, w_ref, b_ref, o_ref):
    acc = jnp.broadcast_to(b_ref[...], (_BG * 8, _DM))
    for j in range(_NTILE):
        xj = x_ref[:, j].reshape(_BG * 8, _DM)
        acc = acc + jnp.dot(xj, w_ref[j], preferred_element_type=jnp.float32)
    o_ref[...] = acc


def _mm(x4d, w4, b2):
    return pl.pallas_call(
        _mm_body,
        grid=(_NGRP // _BG,),
        in_specs=[
            pl.BlockSpec((_BG, _NTILE, 8, _DM), lambda i: (i, 0, 0, 0)),
            pl.BlockSpec((_NTILE, _DM, _DM), lambda i: (0, 0, 0)),
            pl.BlockSpec((1, _DM), lambda i: (0, 0)),
        ],
        out_specs=pl.BlockSpec((_BG * 8, _DM), lambda i: (i, 0)),
        out_shape=jax.ShapeDtypeStruct((_BL, _DM), jnp.float32),
    )(x4d, w4, b2)


def kernel(sample, tables, W, b):
    samp_rs = sample.reshape(_NW, _IDX_W)
    tpad = jnp.pad(tables, ((0, 0), (0, 0), (0, 128 - _DE)))
    tables_flat = tpad.reshape(4 * _NF * _VOCAB, _DE)
    gathered = _sc_gather(
        samp_rs, tables_flat, jnp.asarray(_PERM_NP), jnp.asarray(_OFF_NP)
    )
    x4d = gathered.reshape(_NGRP, _NTILE, 8, _DM)
    w4 = (
        jnp.zeros((_FAN_PAD, _DM), jnp.float32)
        .at[:_FAN_IN]
        .set(W)
        .reshape(_NTILE, _DM, _DM)
    )
    out = _mm(x4d, w4, b.reshape(1, _DM))
    return out.reshape(_B, _L, _DM)
